# baseline (speedup 1.0000x reference)
import functools

import jax
import jax.numpy as jnp
from jax import lax
from jax.experimental import pallas as pl
from jax.experimental.pallas import tpu as pltpu
from jax.experimental.pallas import tpu_sc as plsc

VOCAB = 1000000
D = 64
B = 4096 * 200

_info = plsc.get_sparse_core_info()
NC, NS = _info.num_cores, _info.num_subcores
NW = NC * NS
B_PER_W = B // NW
CHUNK = 128
NCHUNK = B_PER_W // CHUNK


@functools.partial(
    pl.kernel,
    mesh=plsc.VectorSubcoreMesh(core_axis_name="c", subcore_axis_name="s"),
    out_type=jax.ShapeDtypeStruct((B, D), jnp.float32),
    compiler_params=pltpu.CompilerParams(use_tc_tiling_on_sc=False),
    scratch_types=[
        pltpu.VMEM((CHUNK,), jnp.int32),
        pltpu.VMEM((CHUNK, D), jnp.float32),
        pltpu.SemaphoreType.DMA,
    ],
)
def _gather_kernel(x_hbm, table_hbm, out_hbm, idx_v, rows_v, sem):
    wid = lax.axis_index("s") * NC + lax.axis_index("c")
    base = wid * B_PER_W

    def chunk_body(j, _):
        pltpu.sync_copy(x_hbm.at[wid, j], idx_v)
        pltpu.async_copy(table_hbm.at[idx_v], rows_v, sem).wait()
        pltpu.sync_copy(rows_v, out_hbm.at[pl.ds(base + j * CHUNK, CHUNK)])
        return _

    lax.fori_loop(0, NCHUNK, chunk_body, None)


def kernel(x, table):
    x3 = x.reshape(NW, NCHUNK, CHUNK)
    out = _gather_kernel(x3, table)
    return out.reshape(x.shape[0], x.shape[1], D)


# speedup vs baseline: 1.1969x; 1.1969x over previous
import functools

import jax
import jax.numpy as jnp
from jax import lax
from jax.experimental import pallas as pl
from jax.experimental.pallas import tpu as pltpu
from jax.experimental.pallas import tpu_sc as plsc

VOCAB = 1000000
D = 64
B = 4096 * 200

_info = plsc.get_sparse_core_info()
NC, NS = _info.num_cores, _info.num_subcores
NW = NC * NS
B_PER_W = B // NW
CHUNK = 512
NCHUNK = B_PER_W // CHUNK
NBUF = 2
NGRP = NCHUNK // NBUF


@functools.partial(
    pl.kernel,
    mesh=plsc.VectorSubcoreMesh(core_axis_name="c", subcore_axis_name="s"),
    out_type=jax.ShapeDtypeStruct((B, D), jnp.float32),
    compiler_params=pltpu.CompilerParams(use_tc_tiling_on_sc=False),
    scratch_types=[
        pltpu.VMEM((CHUNK,), jnp.int32),
        pltpu.VMEM((CHUNK,), jnp.int32),
        pltpu.VMEM((CHUNK, D), jnp.float32),
        pltpu.VMEM((CHUNK, D), jnp.float32),
        pltpu.SemaphoreType.DMA,
        pltpu.SemaphoreType.DMA,
        pltpu.SemaphoreType.DMA,
        pltpu.SemaphoreType.DMA,
    ],
)
def _gather_kernel(x_hbm, table_hbm, out_hbm,
                   idx0, idx1, rows0, rows1, g0, g1, w0, w1):
    wid = lax.axis_index("s") * NC + lax.axis_index("c")
    base = wid * B_PER_W
    idx = (idx0, idx1)
    rows = (rows0, rows1)
    gsem = (g0, g1)
    wsem = (w0, w1)

    for b in range(NBUF):
        pltpu.sync_copy(x_hbm.at[wid, b], idx[b])
        pltpu.async_copy(table_hbm.at[idx[b]], rows[b], gsem[b])

    def group(g, _):
        for b in range(NBUF):
            j = g * NBUF + b
            pltpu.make_async_copy(table_hbm.at[idx[b]], rows[b], gsem[b]).wait()
            pltpu.async_copy(
                rows[b], out_hbm.at[pl.ds(base + j * CHUNK, CHUNK)], wsem[b])
            jn = j + NBUF
            pltpu.sync_copy(x_hbm.at[wid, jn], idx[b])
            pltpu.make_async_copy(
                rows[b], out_hbm.at[pl.ds(base + j * CHUNK, CHUNK)],
                wsem[b]).wait()
            pltpu.async_copy(table_hbm.at[idx[b]], rows[b], gsem[b])
        return _

    lax.fori_loop(0, NGRP - 1, group, None)

    for b in range(NBUF):
        j = (NGRP - 1) * NBUF + b
        pltpu.make_async_copy(table_hbm.at[idx[b]], rows[b], gsem[b]).wait()
        pltpu.async_copy(
            rows[b], out_hbm.at[pl.ds(base + j * CHUNK, CHUNK)], wsem[b])
    for b in range(NBUF):
        j = (NGRP - 1) * NBUF + b
        pltpu.make_async_copy(
            rows[b], out_hbm.at[pl.ds(base + j * CHUNK, CHUNK)], wsem[b]).wait()


def kernel(x, table):
    x3 = x.reshape(NW, NCHUNK, CHUNK)
    out = _gather_kernel(x3, table)
    return out.reshape(x.shape[0], x.shape[1], D)
